# baseline (speedup 1.0000x reference)
import functools

import jax
import jax.numpy as jnp
from jax.experimental import pallas as pl
from jax.experimental.pallas import tpu as pltpu

BETA = 1.0
MARGIN = 0.5

BLK_B = 128
T_PAD = 64


def _loss_kernel(logits_ref, target_ref, mask_ref, out_ref, *, inv_b):
    blk_b, n = logits_ref.shape
    x = logits_ref[...]
    t = target_ref[...]

    a = mask_ref[:, 0:32, :]
    c = mask_ref[:, 18:50, :]
    wa = pltpu.bitcast(a.reshape(blk_b * 32, n), jnp.int32)
    wc = pltpu.bitcast(c.reshape(blk_b * 32, n), jnp.int32)
    w = (wa | wc).reshape(blk_b, 8, n)
    w = w[:, :4, :] | w[:, 4:, :]
    w = w[:, :2, :] | w[:, 2:, :]
    words = w[:, 0, :] | w[:, 1, :]
    valid = words != 0

    lane = jax.lax.broadcasted_iota(jnp.int32, (blk_b, n), 1)
    is_t = lane == t
    neg = jnp.where(valid & (~is_t), x, jnp.float32(-10000.0))
    neg_max = jnp.max(neg, axis=1)
    pos = jnp.max(jnp.where(is_t, x, jnp.float32(-jnp.inf)), axis=1)

    z = BETA * (neg_max - pos + MARGIN)
    per = jnp.maximum(z, 0.0) + jnp.log1p(jnp.exp(-jnp.abs(z)))
    part = jnp.sum(per) * inv_b

    @pl.when(pl.program_id(0) == 0)
    def _():
        out_ref[0, 0] = jnp.float32(0.0)

    out_ref[0, 0] += part


@jax.jit
def kernel(importance_logits, target_index, person_mask):
    b, n = importance_logits.shape
    _, t_dim, _ = person_mask.shape
    tgt = target_index.astype(jnp.int32).reshape(b, 1)
    mask_i8 = person_mask.astype(jnp.int8)

    grid = (b // BLK_B,)
    out = pl.pallas_call(
        functools.partial(_loss_kernel, inv_b=1.0 / b),
        grid=grid,
        in_specs=[
            pl.BlockSpec((BLK_B, n), lambda i: (i, 0)),
            pl.BlockSpec((BLK_B, 1), lambda i: (i, 0)),
            pl.BlockSpec((BLK_B, t_dim, n), lambda i: (i, 0, 0)),
        ],
        out_specs=pl.BlockSpec(
            (1, 1), lambda i: (0, 0), memory_space=pltpu.SMEM
        ),
        out_shape=jax.ShapeDtypeStruct((1, 1), jnp.float32),
    )(importance_logits, tgt, mask_i8)
    return out[0, 0]


# speedup vs baseline: 1.0563x; 1.0563x over previous
import functools

import jax
import jax.numpy as jnp
from jax.experimental import pallas as pl
from jax.experimental.pallas import tpu as pltpu

BETA = 1.0
MARGIN = 0.5

BLK_B = 128
T_PAD = 64


def _loss_kernel(logits_ref, target_ref, mask_ref, out_ref, *, inv_b):
    blk_b, n = logits_ref.shape
    x = logits_ref[...]
    t = target_ref[...]

    m = mask_ref[...]
    w = pltpu.bitcast(m.reshape(blk_b * T_PAD, n), jnp.int32)
    w = w.reshape(blk_b, T_PAD // 4, n)
    w = w[:, :8, :] | w[:, 8:, :]
    w = w[:, :4, :] | w[:, 4:, :]
    w = w[:, :2, :] | w[:, 2:, :]
    words = w[:, 0, :] | w[:, 1, :]
    valid = words != 0

    lane = jax.lax.broadcasted_iota(jnp.int32, (blk_b, n), 1)
    is_t = lane == t
    neg = jnp.where(valid & (~is_t), x, jnp.float32(-10000.0))
    neg_max = jnp.max(neg, axis=1)
    pos = jnp.max(jnp.where(is_t, x, jnp.float32(-jnp.inf)), axis=1)

    z = BETA * (neg_max - pos + MARGIN)
    per = jnp.maximum(z, 0.0) + jnp.log1p(jnp.exp(-jnp.abs(z)))
    part = jnp.sum(per) * inv_b

    @pl.when(pl.program_id(0) == 0)
    def _():
        out_ref[0, 0] = jnp.float32(0.0)

    out_ref[0, 0] += part


N_CHUNKS = 4


@jax.jit
def kernel(importance_logits, target_index, person_mask):
    b, n = importance_logits.shape
    _, t_dim, _ = person_mask.shape
    tgt = target_index.astype(jnp.int32).reshape(b, 1)

    bc = b // N_CHUNKS
    partials = []
    for c in range(N_CHUNKS):
        sl = slice(c * bc, (c + 1) * bc)
        mask_i8 = jnp.concatenate(
            [person_mask[sl].astype(jnp.int8),
             jnp.zeros((bc, T_PAD - t_dim, n), jnp.int8)],
            axis=1,
        )
        grid = (bc // BLK_B,)
        out = pl.pallas_call(
            functools.partial(_loss_kernel, inv_b=1.0 / b),
            grid=grid,
            in_specs=[
                pl.BlockSpec((BLK_B, n), lambda i: (i, 0)),
                pl.BlockSpec((BLK_B, 1), lambda i: (i, 0)),
                pl.BlockSpec((BLK_B, T_PAD, n), lambda i: (i, 0, 0)),
            ],
            out_specs=pl.BlockSpec(
                (1, 1), lambda i: (0, 0), memory_space=pltpu.SMEM
            ),
            out_shape=jax.ShapeDtypeStruct((1, 1), jnp.float32),
        )(importance_logits[sl], tgt[sl], mask_i8)
        partials.append(out[0, 0])
    return sum(partials)


# speedup vs baseline: 1.3669x; 1.2940x over previous
import functools

import jax
import jax.numpy as jnp
from jax.experimental import pallas as pl
from jax.experimental.pallas import tpu as pltpu

BETA = 1.0
MARGIN = 0.5

BLK_B = 128
T_PAD = 64


def _loss_kernel(logits_ref, target_ref, mask_ref, out_ref, *, inv_b):
    blk_b, n = logits_ref.shape
    x = logits_ref[...]
    t = target_ref[...]

    m = mask_ref[...]
    w = pltpu.bitcast(m.reshape(blk_b * T_PAD, n), jnp.int32)
    w = w.reshape(blk_b, T_PAD // 8, n)
    w = w[:, :4, :] | w[:, 4:, :]
    w = w[:, :2, :] | w[:, 2:, :]
    words = w[:, 0, :] | w[:, 1, :]
    valid = words != 0

    lane = jax.lax.broadcasted_iota(jnp.int32, (blk_b, n), 1)
    is_t = lane == t
    neg = jnp.where(valid & (~is_t), x, jnp.float32(-10000.0))
    neg_max = jnp.max(neg, axis=1)
    pos = jnp.max(jnp.where(is_t, x, jnp.float32(-jnp.inf)), axis=1)

    z = BETA * (neg_max - pos + MARGIN)
    per = jnp.maximum(z, 0.0) + jnp.log1p(jnp.exp(-jnp.abs(z)))
    part = jnp.sum(per) * inv_b

    @pl.when(pl.program_id(0) == 0)
    def _():
        out_ref[0, 0] = jnp.float32(0.0)

    out_ref[0, 0] += part


@jax.jit
def kernel(importance_logits, target_index, person_mask):
    b, n = importance_logits.shape
    _, t_dim, _ = person_mask.shape
    tgt = target_index.astype(jnp.int32).reshape(b, 1)
    mask_i4 = jnp.concatenate(
        [person_mask.astype(jnp.int4),
         jnp.zeros((b, T_PAD - t_dim, n), jnp.int4)],
        axis=1,
    )

    grid = (b // BLK_B,)
    out = pl.pallas_call(
        functools.partial(_loss_kernel, inv_b=1.0 / b),
        grid=grid,
        in_specs=[
            pl.BlockSpec((BLK_B, n), lambda i: (i, 0)),
            pl.BlockSpec((BLK_B, 1), lambda i: (i, 0)),
            pl.BlockSpec((BLK_B, T_PAD, n), lambda i: (i, 0, 0)),
        ],
        out_specs=pl.BlockSpec(
            (1, 1), lambda i: (0, 0), memory_space=pltpu.SMEM
        ),
        out_shape=jax.ShapeDtypeStruct((1, 1), jnp.float32),
    )(importance_logits, tgt, mask_i4)
    return out[0, 0]


# speedup vs baseline: 1.3738x; 1.0050x over previous
import functools

import jax
import jax.numpy as jnp
from jax.experimental import pallas as pl
from jax.experimental.pallas import tpu as pltpu

BETA = 1.0
MARGIN = 0.5

BLK_B = 256
T_PAD = 64


def _loss_kernel(logits_ref, target_ref, mask_ref, out_ref, *, inv_b):
    blk_b, n = logits_ref.shape
    x = logits_ref[...]
    t = target_ref[...]

    m = mask_ref[...]
    w = pltpu.bitcast(m.reshape(blk_b * T_PAD, n), jnp.int32)
    w = w.reshape(blk_b, T_PAD // 8, n)
    w = w[:, :4, :] | w[:, 4:, :]
    w = w[:, :2, :] | w[:, 2:, :]
    words = w[:, 0, :] | w[:, 1, :]
    valid = words != 0

    lane = jax.lax.broadcasted_iota(jnp.int32, (blk_b, n), 1)
    is_t = lane == t
    neg = jnp.where(valid & (~is_t), x, jnp.float32(-10000.0))
    neg_max = jnp.max(neg, axis=1)
    pos = jnp.max(jnp.where(is_t, x, jnp.float32(-jnp.inf)), axis=1)

    z = BETA * (neg_max - pos + MARGIN)
    per = jnp.maximum(z, 0.0) + jnp.log1p(jnp.exp(-jnp.abs(z)))
    part = jnp.sum(per) * inv_b

    @pl.when(pl.program_id(0) == 0)
    def _():
        out_ref[0, 0] = jnp.float32(0.0)

    out_ref[0, 0] += part


@jax.jit
def kernel(importance_logits, target_index, person_mask):
    b, n = importance_logits.shape
    _, t_dim, _ = person_mask.shape
    tgt = target_index.astype(jnp.int32).reshape(b, 1)
    mask_i4 = jnp.concatenate(
        [person_mask.astype(jnp.int4),
         jnp.zeros((b, T_PAD - t_dim, n), jnp.int4)],
        axis=1,
    )

    grid = (b // BLK_B,)
    out = pl.pallas_call(
        functools.partial(_loss_kernel, inv_b=1.0 / b),
        grid=grid,
        in_specs=[
            pl.BlockSpec((BLK_B, n), lambda i: (i, 0)),
            pl.BlockSpec((BLK_B, 1), lambda i: (i, 0)),
            pl.BlockSpec((BLK_B, T_PAD, n), lambda i: (i, 0, 0)),
        ],
        out_specs=pl.BlockSpec(
            (1, 1), lambda i: (0, 0), memory_space=pltpu.SMEM
        ),
        out_shape=jax.ShapeDtypeStruct((1, 1), jnp.float32),
    )(importance_logits, tgt, mask_i4)
    return out[0, 0]
